# baseline (speedup 1.0000x reference)
import jax
import jax.numpy as jnp
from jax import lax
from jax.experimental import pallas as pl
from jax.experimental.pallas import tpu as pltpu
from jax.experimental.pallas import tpu_sc as plsc

_N = 1000
_F = 16
_B = 1024
_ROW = _N * _F

_info = plsc.get_sparse_core_info()
_NC = _info.num_cores
_NS = _info.num_subcores
_NW = _NC * _NS
_BPW = _B // _NW


def _broadcast_body(table_hbm, out_hbm, buf, sem):
    wid = lax.axis_index("s") * _NC + lax.axis_index("c")
    base = wid * _BPW * _ROW
    pltpu.sync_copy(table_hbm, buf)
    copies = [
        pltpu.make_async_copy(buf, out_hbm.at[pl.ds(base + i * _ROW, _ROW)], sem)
        for i in range(_BPW)
    ]
    for c in copies:
        c.start()
    for c in copies:
        c.wait()


@jax.jit
def kernel(x, embed_weight):
    del x
    table = embed_weight.reshape(_ROW)
    mesh = plsc.VectorSubcoreMesh(core_axis_name="c", subcore_axis_name="s")
    out = pl.kernel(
        _broadcast_body,
        out_type=jax.ShapeDtypeStruct((_B * _ROW,), jnp.float32),
        mesh=mesh,
        scratch_types=[
            pltpu.VMEM((_ROW,), jnp.float32),
            pltpu.SemaphoreType.DMA,
        ],
    )(table)
    return out.reshape(_B, _N, _F)
